# baseline (speedup 1.0000x reference)
import functools

import jax
import jax.numpy as jnp
import numpy as np
from jax.experimental import pallas as pl
from jax.experimental.pallas import tpu as pltpu

L_IN = 420
KW = 28
L_OUT = 15
C_OUT = 32
HID = 32
F = L_OUT * C_OUT
P0 = 8
P1 = L_OUT - P0
K0 = P0 * KW
K1 = P1 * KW
N0 = P0 * C_OUT
N1 = P1 * C_OUT
NP = 256


def _net_kernel(x_ref, w0_ref, bb0_ref, w1a_ref,
                w1c_ref, bb1_ref, w1b_ref,
                b1_ref, w2_ref, b2_ref, w3_ref, b3_ref, out_ref):
    xb = x_ref[...].astype(jnp.bfloat16)
    zero = jnp.bfloat16(0.0)
    h0 = jnp.dot(xb[:, :K0], w0_ref[...],
                 preferred_element_type=jnp.float32).astype(jnp.bfloat16)
    h0 = jnp.maximum(h0 + bb0_ref[...], zero)
    h1 = jnp.dot(xb[:, K0:], w1c_ref[...],
                 preferred_element_type=jnp.float32).astype(jnp.bfloat16)
    h1 = jnp.maximum(h1 + bb1_ref[...], zero)
    y = (jnp.dot(h0, w1a_ref[...], preferred_element_type=jnp.float32)
         + jnp.dot(h1, w1b_ref[...], preferred_element_type=jnp.float32))
    yt = jnp.transpose(y.astype(jnp.bfloat16))
    yt = jnp.maximum(yt + b1_ref[...], zero)
    zt = jnp.dot(w2_ref[...], yt, preferred_element_type=jnp.float32)
    zt = jnp.maximum(zt.astype(jnp.bfloat16) + b2_ref[...], zero)
    logit = jnp.dot(w3_ref[...], zt,
                    preferred_element_type=jnp.float32) + b3_ref[...]
    out_ref[...] = jax.nn.sigmoid(logit)


def _prep_weights(wc, bc, w1, b1, w2, b2, w3, b3):
    wct = jnp.transpose(wc[:, 0, :]).astype(jnp.float32)
    def blockdiag(npos):
        eye = jnp.eye(npos, dtype=jnp.float32)
        return jnp.einsum('lm,kc->lkmc', eye, wct).reshape(npos * KW, npos * C_OUT)

    w0 = blockdiag(P0).astype(jnp.bfloat16)
    w1c = jnp.pad(blockdiag(P1), ((0, 0), (0, NP - N1))).astype(jnp.bfloat16)
    bb0 = jnp.tile(bc, P0).reshape(1, N0).astype(jnp.bfloat16)
    bb1 = jnp.pad(jnp.tile(bc, P1), (0, NP - N1)).reshape(1, NP).astype(jnp.bfloat16)
    w1r = jnp.transpose(w1.reshape(HID, C_OUT, L_OUT), (2, 1, 0)).reshape(F, HID)
    w1a = w1r[:N0].astype(jnp.bfloat16)
    w1b = jnp.pad(w1r[N0:], ((0, NP - N1), (0, 0))).astype(jnp.bfloat16)
    b1c = b1.reshape(HID, 1).astype(jnp.bfloat16)
    w2n = w2.astype(jnp.bfloat16)
    b2c = b2.reshape(HID, 1).astype(jnp.bfloat16)
    w3n = w3.astype(jnp.bfloat16)
    b3r = b3.reshape(1, 1)
    return w0, bb0, w1a, w1c, bb1, w1b, b1c, w2n, b2c, w3n, b3r


@jax.jit
def kernel(x, wc, bc, w1, b1, w2, b2, w3, b3):
    B = x.shape[0]
    weights = _prep_weights(wc, bc, w1, b1, w2, b2, w3, b3)

    x_flat = x.reshape(B, L_IN)

    TB = min(1024, max(8, ((B + 7) // 8) * 8))
    Bp = ((B + TB - 1) // TB) * TB
    if Bp != B:
        x_flat = jnp.pad(x_flat, ((0, Bp - B), (0, 0)))
    grid = (Bp // TB,)

    def wspec(shape):
        return pl.BlockSpec(shape, lambda i: (0, 0))

    out = pl.pallas_call(
        _net_kernel,
        out_shape=jax.ShapeDtypeStruct((1, Bp), jnp.float32),
        grid=grid,
        in_specs=[pl.BlockSpec((TB, L_IN), lambda i: (i, 0)),
                  wspec((K0, N0)), wspec((1, N0)), wspec((N0, HID)),
                  wspec((K1, NP)), wspec((1, NP)), wspec((NP, HID)),
                  wspec((HID, 1)), wspec((HID, HID)), wspec((HID, 1)),
                  wspec((1, HID)), wspec((1, 1))],
        out_specs=pl.BlockSpec((1, TB), lambda i: (0, i)),
        compiler_params=pltpu.CompilerParams(dimension_semantics=("arbitrary",)),
    )(x_flat, *weights)

    return out.reshape(Bp, 1)[:B]


# speedup vs baseline: 1.0644x; 1.0644x over previous
import functools

import jax
import jax.numpy as jnp
import numpy as np
from jax.experimental import pallas as pl
from jax.experimental.pallas import tpu as pltpu

L_IN = 420
KW = 28
L_OUT = 15
C_OUT = 32
HID = 32
F = L_OUT * C_OUT
P0 = 8
P1 = L_OUT - P0
K0 = P0 * KW
K1 = P1 * KW
N0 = P0 * C_OUT
N1 = P1 * C_OUT
NP = 256


def _net_kernel(x_ref, w0_ref, bb0_ref, w1a_ref,
                w1c_ref, bb1_ref, w1b_ref,
                b1_ref, w2_ref, b2_ref, w3_ref, b3_ref, out_ref):
    xb = x_ref[...].astype(jnp.bfloat16)
    zero = jnp.bfloat16(0.0)
    h0 = jnp.dot(xb[:, :K0], w0_ref[...],
                 preferred_element_type=jnp.float32).astype(jnp.bfloat16)
    h0 = jnp.maximum(h0 + bb0_ref[...], zero)
    h1 = jnp.dot(xb[:, K0:], w1c_ref[...],
                 preferred_element_type=jnp.float32).astype(jnp.bfloat16)
    h1 = jnp.maximum(h1 + bb1_ref[...], zero)
    y = (jnp.dot(h0, w1a_ref[...], preferred_element_type=jnp.float32)
         + jnp.dot(h1, w1b_ref[...], preferred_element_type=jnp.float32))
    yt = jnp.transpose(y.astype(jnp.bfloat16))
    yt = jnp.maximum(yt + b1_ref[...], zero)
    zt = jnp.dot(w2_ref[...], yt, preferred_element_type=jnp.float32)
    zt = jnp.maximum(zt.astype(jnp.bfloat16) + b2_ref[...], zero)
    logit = jnp.dot(w3_ref[...], zt,
                    preferred_element_type=jnp.float32) + b3_ref[...]
    out_ref[...] = jax.nn.sigmoid(logit)


def _prep_weights(wc, bc, w1, b1, w2, b2, w3, b3):
    wct = jnp.transpose(wc[:, 0, :]).astype(jnp.float32)
    def blockdiag(npos):
        eye = jnp.eye(npos, dtype=jnp.float32)
        return jnp.einsum('lm,kc->lkmc', eye, wct).reshape(npos * KW, npos * C_OUT)

    w0 = blockdiag(P0).astype(jnp.bfloat16)
    w1c = jnp.pad(blockdiag(P1), ((0, 0), (0, NP - N1))).astype(jnp.bfloat16)
    bb0 = jnp.tile(bc, P0).reshape(1, N0).astype(jnp.bfloat16)
    bb1 = jnp.pad(jnp.tile(bc, P1), (0, NP - N1)).reshape(1, NP).astype(jnp.bfloat16)
    w1r = jnp.transpose(w1.reshape(HID, C_OUT, L_OUT), (2, 1, 0)).reshape(F, HID)
    w1a = w1r[:N0].astype(jnp.bfloat16)
    w1b = jnp.pad(w1r[N0:], ((0, NP - N1), (0, 0))).astype(jnp.bfloat16)
    b1c = b1.reshape(HID, 1).astype(jnp.bfloat16)
    w2n = w2.astype(jnp.bfloat16)
    b2c = b2.reshape(HID, 1).astype(jnp.bfloat16)
    w3n = w3.astype(jnp.bfloat16)
    b3r = b3.reshape(1, 1)
    return w0, bb0, w1a, w1c, bb1, w1b, b1c, w2n, b2c, w3n, b3r


@jax.jit
def kernel(x, wc, bc, w1, b1, w2, b2, w3, b3):
    B = x.shape[0]
    weights = _prep_weights(wc, bc, w1, b1, w2, b2, w3, b3)

    x_flat = x.reshape(B, L_IN)

    TB = min(4096, max(8, ((B + 7) // 8) * 8))
    Bp = ((B + TB - 1) // TB) * TB
    if Bp != B:
        x_flat = jnp.pad(x_flat, ((0, Bp - B), (0, 0)))
    grid = (Bp // TB,)

    def wspec(shape):
        return pl.BlockSpec(shape, lambda i: (0, 0))

    out = pl.pallas_call(
        _net_kernel,
        out_shape=jax.ShapeDtypeStruct((1, Bp), jnp.float32),
        grid=grid,
        in_specs=[pl.BlockSpec((TB, L_IN), lambda i: (i, 0)),
                  wspec((K0, N0)), wspec((1, N0)), wspec((N0, HID)),
                  wspec((K1, NP)), wspec((1, NP)), wspec((NP, HID)),
                  wspec((HID, 1)), wspec((HID, HID)), wspec((HID, 1)),
                  wspec((1, HID)), wspec((1, 1))],
        out_specs=pl.BlockSpec((1, TB), lambda i: (0, i)),
        compiler_params=pltpu.CompilerParams(dimension_semantics=("arbitrary",)),
    )(x_flat, *weights)

    return out.reshape(Bp, 1)[:B]
